# baseline (speedup 1.0000x reference)
import functools

import jax
import jax.numpy as jnp
from jax import lax
from jax.experimental import pallas as pl
from jax.experimental.pallas import tpu as pltpu
from jax.experimental.pallas import tpu_sc as plsc

N = 10000
D = 128
AD = 17
N_PAD = 10240
NC = 2
NS = 16
NW = NC * NS
GRP = 64
G_PER_W = 160
G_SEG = 40
N_SEG = G_PER_W // G_SEG
E_PAD = NW * G_PER_W * GRP
ROWS_PER_SUB = N_PAD // NS

_mesh = plsc.VectorSubcoreMesh(core_axis_name="c", subcore_axis_name="s")


@functools.partial(
    pl.kernel,
    out_type=jax.ShapeDtypeStruct((NC, N_PAD, D), jnp.float32),
    mesh=_mesh,
    scratch_types=[
        pltpu.VMEM_SHARED((N_PAD, D), jnp.float32),
        pltpu.VMEM((G_SEG, GRP), jnp.int32),
        pltpu.VMEM((G_SEG, GRP), jnp.int32),
        pltpu.VMEM((GRP, D), jnp.float32),
        pltpu.VMEM((GRP, D), jnp.float32),
        pltpu.VMEM((GRP, D), jnp.float32),
        pltpu.VMEM((GRP, D), jnp.float32),
        pltpu.SemaphoreType.DMA,
        pltpu.SemaphoreType.DMA,
        pltpu.SemaphoreType.DMA,
        pltpu.SemaphoreType.DMA,
        pltpu.SemaphoreType.DMA,
        pltpu.SemaphoreType.DMA,
        pltpu.SemaphoreType.DMA,
        pltpu.SemaphoreType.DMA,
    ],
)
def _sc_segment_sum(h_hbm, src_hbm, dst_hbm, zeros_hbm, out_hbm,
                    agg_sh, src_v, dst_v, b0, b1, b2, b3,
                    g0, g1, g2, g3, s0, s1, s2, s3):
    c = lax.axis_index("c")
    s = lax.axis_index("s")
    w = s * NC + c
    rows = pl.ds(s * ROWS_PER_SUB, ROWS_PER_SUB)
    pltpu.sync_copy(zeros_hbm.at[rows], agg_sh.at[rows])
    plsc.subcore_barrier()

    bufs = (b0, b1, b2, b3)
    gsem = (g0, g1, g2, g3)
    ssem = (s0, s1, s2, s3)

    @pl.loop(0, N_SEG)
    def _(seg):
        gbase = w * G_PER_W + seg * G_SEG
        pltpu.sync_copy(src_hbm.at[pl.ds(gbase, G_SEG)], src_v)
        pltpu.sync_copy(dst_hbm.at[pl.ds(gbase, G_SEG)], dst_v)
        for b in range(4):
            pltpu.async_copy(h_hbm.at[src_v.at[b]], bufs[b], gsem[b])

        @pl.loop(0, G_SEG, step=4)
        def _(g):
            for b in range(4):
                pltpu.make_async_copy(
                    h_hbm.at[src_v.at[g + b]], bufs[b], gsem[b]).wait()
                pltpu.async_copy(bufs[b], agg_sh.at[dst_v.at[g + b]],
                                 ssem[b], add=True)
            for b in range(4):
                pltpu.make_async_copy(
                    bufs[b], agg_sh.at[dst_v.at[g + b]], ssem[b]).wait()

                @pl.when(g + 4 + b < G_SEG)
                def _():
                    pltpu.async_copy(
                        h_hbm.at[src_v.at[g + 4 + b]], bufs[b], gsem[b])

    plsc.subcore_barrier()
    pltpu.sync_copy(agg_sh.at[rows], out_hbm.at[c].at[rows])


_BLK = 400


def _linear_relu_body(x_ref, w_ref, b_ref, o_ref):
    acc = jnp.dot(x_ref[...], w_ref[...], preferred_element_type=jnp.float32)
    o_ref[...] = jnp.maximum(acc + b_ref[...], 0.0)


def _tc_linear_relu(x, w, b):
    return pl.pallas_call(
        _linear_relu_body,
        grid=(N // _BLK,),
        in_specs=[
            pl.BlockSpec((_BLK, D), lambda i: (i, 0)),
            pl.BlockSpec((D, D), lambda i: (0, 0)),
            pl.BlockSpec((1, D), lambda i: (0, 0)),
        ],
        out_specs=pl.BlockSpec((_BLK, D), lambda i: (i, 0)),
        out_shape=jax.ShapeDtypeStruct((N, D), jnp.float32),
    )(x, w, b.reshape(1, D))


def _combine_body(p_ref, deg_ref, w_ref, b_ref, o_ref):
    p = p_ref[0] + p_ref[1]
    d = deg_ref[0, :, 0:1] + deg_ref[1, :, 0:1]
    inv = 1.0 / jnp.maximum(d, 1.0)
    acc = jnp.dot(p * inv, w_ref[...], preferred_element_type=jnp.float32)
    o_ref[...] = jnp.maximum(acc + b_ref[...], 0.0)


def _tc_combine(parts, deg, w, b):
    return pl.pallas_call(
        _combine_body,
        grid=(N // _BLK,),
        in_specs=[
            pl.BlockSpec((NC, _BLK, D), lambda i: (0, i, 0)),
            pl.BlockSpec((NC, _BLK, D), lambda i: (0, i, 0)),
            pl.BlockSpec((D, D), lambda i: (0, 0)),
            pl.BlockSpec((1, D), lambda i: (0, 0)),
        ],
        out_specs=pl.BlockSpec((_BLK, D), lambda i: (i, 0)),
        out_shape=jax.ShapeDtypeStruct((N, D), jnp.float32),
    )(parts, deg, w, b.reshape(1, D))


def _lgamma(z):
    w = z + 7.0
    lprod = (jnp.log(z) + jnp.log(z + 1.0) + jnp.log(z + 2.0)
             + jnp.log(z + 3.0) + jnp.log(z + 4.0) + jnp.log(z + 5.0)
             + jnp.log(z + 6.0))
    wi = 1.0 / w
    wi2 = wi * wi
    stir = (w - 0.5) * jnp.log(w) - w + 0.91893853320467274178
    corr = wi * (1.0 / 12.0 - wi2 * (1.0 / 360.0 - wi2 * (1.0 / 1260.0)))
    return stir + corr - lprod


def _softplus(x):
    return jnp.maximum(x, 0.0) + jnp.log(1.0 + jnp.exp(-jnp.abs(x)))


def _heads_body(h_ref, wa1_ref, ba1_ref, wa2_ref, ba2_ref,
                wc1_ref, bc1_ref, wc2_ref, bc2_ref,
                act_ref, lp_ref, val_ref):
    h = h_ref[...]
    pooled = jnp.sum(h, axis=0, keepdims=True) * (1.0 / N)
    dev = h[0:1000]
    a1 = jnp.maximum(
        jnp.dot(dev, wa1_ref[...], preferred_element_type=jnp.float32)
        + ba1_ref[...], 0.0)
    raw = jnp.dot(a1, wa2_ref[...], preferred_element_type=jnp.float32) + ba2_ref[...]
    conc = _softplus(raw) + 1.0
    csum = jnp.sum(conc, axis=-1, keepdims=True)
    action = conc / csum
    act_ref[...] = action
    lp = (jnp.sum((conc - 1.0) * jnp.log(action))
          + jnp.sum(_lgamma(csum)) - jnp.sum(_lgamma(conc)))
    lp_ref[...] = jnp.reshape(lp, (1, 1))
    v1 = jnp.maximum(
        jnp.dot(pooled, wc1_ref[...], preferred_element_type=jnp.float32)
        + bc1_ref[...], 0.0)
    val_ref[...] = (jnp.dot(v1, wc2_ref[...], preferred_element_type=jnp.float32)
                    + bc2_ref[...])


def _tc_heads(h, wa1, ba1, wa2, ba2, wc1, bc1, wc2, bc2):
    return pl.pallas_call(
        _heads_body,
        out_shape=[
            jax.ShapeDtypeStruct((1000, AD), jnp.float32),
            jax.ShapeDtypeStruct((1, 1), jnp.float32),
            jax.ShapeDtypeStruct((1, 1), jnp.float32),
        ],
    )(h, wa1, ba1.reshape(1, D), wa2, ba2.reshape(1, AD),
      wc1, bc1.reshape(1, D), wc2, bc2.reshape(1, 1))


def kernel(x, edge_index, W_in, b_in, W_layers, b_layers,
           W_a1, b_a1, W_a2, b_a2, W_c1, b_c1, W_c2, b_c2):
    e = edge_index.shape[1]
    pad = E_PAD - e
    src_p = jnp.concatenate(
        [edge_index[0], jnp.zeros((pad,), jnp.int32)]).reshape(E_PAD // GRP, GRP)
    dst_p = jnp.concatenate(
        [edge_index[1], jnp.full((pad,), N, jnp.int32)]).reshape(E_PAD // GRP, GRP)
    zeros_d = jnp.zeros((N_PAD, D), jnp.float32)
    ones_n = jnp.ones((N, D), jnp.float32)

    deg = _sc_segment_sum(ones_n, src_p, dst_p, zeros_d)
    h = _tc_linear_relu(x, W_in, b_in)
    for l in range(3):
        parts = _sc_segment_sum(h, src_p, dst_p, zeros_d)
        h = _tc_combine(parts, deg, W_layers[l], b_layers[l])
    action, lp, val = _tc_heads(h, W_a1, b_a1, W_a2, b_a2,
                                W_c1, b_c1, W_c2, b_c2)
    return action, lp.reshape(()), val.reshape(())


# speedup vs baseline: 1.3728x; 1.3728x over previous
import dataclasses
import functools

import jax
import jax.numpy as jnp
from jax import lax
from jax.experimental import pallas as pl
from jax.experimental.pallas import tpu as pltpu
from jax.experimental.pallas import tpu_sc as plsc

N = 10000
D = 128
AD = 17
N_PAD = 10240
NC = 2
NS = 16
NW = NC * NS
GRP = 64
G_PER_W = 160
G_SEG = 40
N_SEG = G_PER_W // G_SEG
E_PAD = NW * G_PER_W * GRP
ROWS_PER_SUB = N_PAD // NS

_mesh = plsc.VectorSubcoreMesh(core_axis_name="c", subcore_axis_name="s")


@functools.partial(
    pl.kernel,
    out_type=jax.ShapeDtypeStruct((NC, N_PAD, D), jnp.float32),
    mesh=_mesh,
    scratch_types=[
        pltpu.VMEM_SHARED((N_PAD, D), jnp.float32),
        pltpu.VMEM((G_SEG, GRP), jnp.int32),
        pltpu.VMEM((G_SEG, GRP), jnp.int32),
        pltpu.VMEM((GRP, D), jnp.float32),
        pltpu.VMEM((GRP, D), jnp.float32),
        pltpu.SemaphoreType.DMA,
        pltpu.SemaphoreType.DMA,
    ],
)
def _sc_segment_sum(h_hbm, src_hbm, dst_hbm, zeros_hbm, out_hbm,
                    agg_sh, src_v, dst_v, buf0, buf1, gs0, gs1):
    c = lax.axis_index("c")
    s = lax.axis_index("s")
    w = s * NC + c
    rows = pl.ds(s * ROWS_PER_SUB, ROWS_PER_SUB)
    pltpu.sync_copy(zeros_hbm.at[rows], agg_sh.at[rows])
    plsc.subcore_barrier()

    @pl.loop(0, N_SEG)
    def _(seg):
        gbase = w * G_PER_W + seg * G_SEG
        pltpu.sync_copy(src_hbm.at[pl.ds(gbase, G_SEG)], src_v)
        pltpu.sync_copy(dst_hbm.at[pl.ds(gbase, G_SEG)], dst_v)
        pltpu.async_copy(h_hbm.at[src_v.at[0]], buf0, gs0)
        pltpu.async_copy(h_hbm.at[src_v.at[1]], buf1, gs1)

        @pl.loop(0, G_SEG, step=2)
        def _(g):
            pltpu.make_async_copy(h_hbm.at[src_v.at[g]], buf0, gs0).wait()
            pltpu.sync_copy(buf0, agg_sh.at[dst_v.at[g]], add=True)

            @pl.when(g + 2 < G_SEG)
            def _():
                pltpu.async_copy(h_hbm.at[src_v.at[g + 2]], buf0, gs0)

            pltpu.make_async_copy(h_hbm.at[src_v.at[g + 1]], buf1, gs1).wait()
            pltpu.sync_copy(buf1, agg_sh.at[dst_v.at[g + 1]], add=True)

            @pl.when(g + 3 < G_SEG)
            def _():
                pltpu.async_copy(h_hbm.at[src_v.at[g + 3]], buf1, gs1)

    plsc.subcore_barrier()
    pltpu.sync_copy(agg_sh.at[rows], out_hbm.at[c].at[rows])


HIST_ROWS = N_PAD // D

_sc_cp = pltpu.CompilerParams()
if "needs_layout_passes" in pltpu.CompilerParams.__dataclass_fields__:
    _sc_cp = dataclasses.replace(_sc_cp, needs_layout_passes=False)


@functools.partial(
    pl.kernel,
    out_type=jax.ShapeDtypeStruct((NC, HIST_ROWS, D), jnp.float32),
    mesh=_mesh,
    compiler_params=_sc_cp,
    scratch_types=[
        pltpu.VMEM_SHARED((HIST_ROWS, D), jnp.float32),
        pltpu.VMEM((HIST_ROWS, D), jnp.float32),
        pltpu.VMEM((G_SEG, GRP), jnp.int32),
        pltpu.VMEM((HIST_ROWS,), jnp.int32),
    ],
)
def _sc_degree(dst_hbm, zeros_hbm, out_hbm, deg_sh, hist_v, dst_v, idx_v):
    c = lax.axis_index("c")
    s = lax.axis_index("s")
    w = s * NC + c
    zrows = pl.ds(s * 8, 8)

    @pl.when(s < HIST_ROWS // 8)
    def _():
        pltpu.sync_copy(zeros_hbm.at[zrows], deg_sh.at[zrows])

    pltpu.sync_copy(zeros_hbm.at[pl.ds(0, HIST_ROWS)], hist_v)

    @pl.loop(0, HIST_ROWS, step=16)
    def _(i):
        idx_v[pl.ds(i, 16)] = lax.iota(jnp.int32, 16) + i

    plsc.subcore_barrier()

    ones16 = jnp.ones((16,), jnp.float32)

    @pl.loop(0, N_SEG)
    def _(seg):
        pltpu.sync_copy(dst_hbm.at[pl.ds(w * G_PER_W + seg * G_SEG, G_SEG)],
                        dst_v)

        @pl.loop(0, G_SEG)
        def _(g):
            @pl.loop(0, GRP, step=16)
            def _(j):
                d16 = dst_v[g, pl.ds(j, 16)]
                plsc.addupdate_scatter(
                    hist_v,
                    [jnp.right_shift(d16, 7), jnp.bitwise_and(d16, 127)],
                    ones16)

    pltpu.sync_copy(hist_v, deg_sh.at[idx_v], add=True)
    plsc.subcore_barrier()

    @pl.when(s < HIST_ROWS // 8)
    def _():
        pltpu.sync_copy(deg_sh.at[zrows], out_hbm.at[c].at[zrows])


def _invdeg_body(deg_ref, o_ref):
    d = deg_ref[0] + deg_ref[1]
    o_ref[...] = 1.0 / jnp.maximum(d, 1.0)


def _tc_invdeg(deg):
    return pl.pallas_call(
        _invdeg_body,
        out_shape=jax.ShapeDtypeStruct((HIST_ROWS, D), jnp.float32),
    )(deg)


_BLK = 400


def _linear_relu_body(x_ref, w_ref, b_ref, o_ref):
    acc = jnp.dot(x_ref[...], w_ref[...], preferred_element_type=jnp.float32)
    o_ref[...] = jnp.maximum(acc + b_ref[...], 0.0)


def _tc_linear_relu(x, w, b):
    return pl.pallas_call(
        _linear_relu_body,
        grid=(N // _BLK,),
        in_specs=[
            pl.BlockSpec((_BLK, D), lambda i: (i, 0)),
            pl.BlockSpec((D, D), lambda i: (0, 0)),
            pl.BlockSpec((1, D), lambda i: (0, 0)),
        ],
        out_specs=pl.BlockSpec((_BLK, D), lambda i: (i, 0)),
        out_shape=jax.ShapeDtypeStruct((N, D), jnp.float32),
    )(x, w, b.reshape(1, D))


def _combine_body(p_ref, inv_ref, w_ref, b_ref, o_ref):
    p = p_ref[0] + p_ref[1]
    acc = jnp.dot(p * inv_ref[...], w_ref[...],
                  preferred_element_type=jnp.float32)
    o_ref[...] = jnp.maximum(acc + b_ref[...], 0.0)


def _tc_combine(parts, inv_deg, w, b):
    return pl.pallas_call(
        _combine_body,
        grid=(N // _BLK,),
        in_specs=[
            pl.BlockSpec((NC, _BLK, D), lambda i: (0, i, 0)),
            pl.BlockSpec((_BLK, 1), lambda i: (i, 0)),
            pl.BlockSpec((D, D), lambda i: (0, 0)),
            pl.BlockSpec((1, D), lambda i: (0, 0)),
        ],
        out_specs=pl.BlockSpec((_BLK, D), lambda i: (i, 0)),
        out_shape=jax.ShapeDtypeStruct((N, D), jnp.float32),
    )(parts, inv_deg, w, b.reshape(1, D))


def _lgamma(z):
    w = z + 7.0
    lprod = (jnp.log(z) + jnp.log(z + 1.0) + jnp.log(z + 2.0)
             + jnp.log(z + 3.0) + jnp.log(z + 4.0) + jnp.log(z + 5.0)
             + jnp.log(z + 6.0))
    wi = 1.0 / w
    wi2 = wi * wi
    stir = (w - 0.5) * jnp.log(w) - w + 0.91893853320467274178
    corr = wi * (1.0 / 12.0 - wi2 * (1.0 / 360.0 - wi2 * (1.0 / 1260.0)))
    return stir + corr - lprod


def _softplus(x):
    return jnp.maximum(x, 0.0) + jnp.log(1.0 + jnp.exp(-jnp.abs(x)))


def _heads_body(h_ref, wa1_ref, ba1_ref, wa2_ref, ba2_ref,
                wc1_ref, bc1_ref, wc2_ref, bc2_ref,
                act_ref, lp_ref, val_ref):
    h = h_ref[...]
    pooled = jnp.sum(h, axis=0, keepdims=True) * (1.0 / N)
    dev = h[0:1000]
    a1 = jnp.maximum(
        jnp.dot(dev, wa1_ref[...], preferred_element_type=jnp.float32)
        + ba1_ref[...], 0.0)
    raw = jnp.dot(a1, wa2_ref[...], preferred_element_type=jnp.float32) + ba2_ref[...]
    conc = _softplus(raw) + 1.0
    csum = jnp.sum(conc, axis=-1, keepdims=True)
    action = conc / csum
    act_ref[...] = action
    lp = (jnp.sum((conc - 1.0) * jnp.log(action))
          + jnp.sum(_lgamma(csum)) - jnp.sum(_lgamma(conc)))
    lp_ref[...] = jnp.reshape(lp, (1, 1))
    v1 = jnp.maximum(
        jnp.dot(pooled, wc1_ref[...], preferred_element_type=jnp.float32)
        + bc1_ref[...], 0.0)
    val_ref[...] = (jnp.dot(v1, wc2_ref[...], preferred_element_type=jnp.float32)
                    + bc2_ref[...])


def _tc_heads(h, wa1, ba1, wa2, ba2, wc1, bc1, wc2, bc2):
    return pl.pallas_call(
        _heads_body,
        out_shape=[
            jax.ShapeDtypeStruct((1000, AD), jnp.float32),
            jax.ShapeDtypeStruct((1, 1), jnp.float32),
            jax.ShapeDtypeStruct((1, 1), jnp.float32),
        ],
    )(h, wa1, ba1.reshape(1, D), wa2, ba2.reshape(1, AD),
      wc1, bc1.reshape(1, D), wc2, bc2.reshape(1, 1))


def kernel(x, edge_index, W_in, b_in, W_layers, b_layers,
           W_a1, b_a1, W_a2, b_a2, W_c1, b_c1, W_c2, b_c2):
    e = edge_index.shape[1]
    pad = E_PAD - e
    src_p = jnp.concatenate(
        [edge_index[0], jnp.zeros((pad,), jnp.int32)]).reshape(E_PAD // GRP, GRP)
    dst_p = jnp.concatenate(
        [edge_index[1], jnp.full((pad,), N, jnp.int32)]).reshape(E_PAD // GRP, GRP)
    zeros_d = jnp.zeros((N_PAD, D), jnp.float32)

    deg = _sc_degree(dst_p, zeros_d)
    inv_deg = _tc_invdeg(deg).reshape(N_PAD, 1)
    h = _tc_linear_relu(x, W_in, b_in)
    for l in range(3):
        parts = _sc_segment_sum(h, src_p, dst_p, zeros_d)
        h = _tc_combine(parts, inv_deg[:N], W_layers[l], b_layers[l])
    action, lp, val = _tc_heads(h, W_a1, b_a1, W_a2, b_a2,
                                W_c1, b_c1, W_c2, b_c2)
    return action, lp.reshape(()), val.reshape(())


# speedup vs baseline: 1.4624x; 1.0653x over previous
import dataclasses
import functools

import jax
import jax.numpy as jnp
from jax import lax
from jax.experimental import pallas as pl
from jax.experimental.pallas import tpu as pltpu
from jax.experimental.pallas import tpu_sc as plsc

N = 10000
D = 128
AD = 17
N_PAD = 10240
NC = 2
NS = 16
NW = NC * NS
GRP = 64
G_PER_W = 160
G_SEG = 40
N_SEG = G_PER_W // G_SEG
HEAVY_CORE = 0
G_HEAVY = 240
G_LIGHT = 2 * G_PER_W - G_HEAVY
NSEG_HEAVY = G_HEAVY // G_SEG
NSEG_LIGHT = G_LIGHT // G_SEG
E_PAD = NW * G_PER_W * GRP
ROWS_PER_SUB = N_PAD // NS

_mesh = plsc.VectorSubcoreMesh(core_axis_name="c", subcore_axis_name="s")


@functools.partial(
    pl.kernel,
    out_type=jax.ShapeDtypeStruct((NC, N_PAD, D), jnp.float32),
    mesh=_mesh,
    scratch_types=[
        pltpu.VMEM_SHARED((N_PAD, D), jnp.float32),
        pltpu.VMEM((G_SEG, GRP), jnp.int32),
        pltpu.VMEM((G_SEG, GRP), jnp.int32),
        pltpu.VMEM((GRP, D), jnp.float32),
        pltpu.VMEM((GRP, D), jnp.float32),
        pltpu.SemaphoreType.DMA,
        pltpu.SemaphoreType.DMA,
    ],
)
def _sc_segment_sum(h_hbm, src_hbm, dst_hbm, zeros_hbm, out_hbm,
                    agg_sh, src_v, dst_v, buf0, buf1, gs0, gs1):
    c = lax.axis_index("c")
    s = lax.axis_index("s")
    heavy = c == HEAVY_CORE
    base = jnp.where(heavy, s * G_HEAVY, NS * G_HEAVY + s * G_LIGHT)
    rows = pl.ds(s * ROWS_PER_SUB, ROWS_PER_SUB)
    pltpu.sync_copy(zeros_hbm.at[rows], agg_sh.at[rows])
    plsc.subcore_barrier()

    @pl.loop(0, NSEG_HEAVY)
    def _(seg):
      @pl.when(heavy | (seg < NSEG_LIGHT))
      def _():
        gbase = base + seg * G_SEG
        pltpu.sync_copy(src_hbm.at[pl.ds(gbase, G_SEG)], src_v)
        pltpu.sync_copy(dst_hbm.at[pl.ds(gbase, G_SEG)], dst_v)
        pltpu.async_copy(h_hbm.at[src_v.at[0]], buf0, gs0)
        pltpu.async_copy(h_hbm.at[src_v.at[1]], buf1, gs1)

        @pl.loop(0, G_SEG, step=2)
        def _(g):
            pltpu.make_async_copy(h_hbm.at[src_v.at[g]], buf0, gs0).wait()
            pltpu.sync_copy(buf0, agg_sh.at[dst_v.at[g]], add=True)

            @pl.when(g + 2 < G_SEG)
            def _():
                pltpu.async_copy(h_hbm.at[src_v.at[g + 2]], buf0, gs0)

            pltpu.make_async_copy(h_hbm.at[src_v.at[g + 1]], buf1, gs1).wait()
            pltpu.sync_copy(buf1, agg_sh.at[dst_v.at[g + 1]], add=True)

            @pl.when(g + 3 < G_SEG)
            def _():
                pltpu.async_copy(h_hbm.at[src_v.at[g + 3]], buf1, gs1)

    plsc.subcore_barrier()
    pltpu.sync_copy(agg_sh.at[rows], out_hbm.at[c].at[rows])


HIST_ROWS = N_PAD // D

_sc_cp = pltpu.CompilerParams()
if "needs_layout_passes" in pltpu.CompilerParams.__dataclass_fields__:
    _sc_cp = dataclasses.replace(_sc_cp, needs_layout_passes=False)


@functools.partial(
    pl.kernel,
    out_type=jax.ShapeDtypeStruct((NC, HIST_ROWS, D), jnp.float32),
    mesh=_mesh,
    compiler_params=_sc_cp,
    scratch_types=[
        pltpu.VMEM_SHARED((HIST_ROWS, D), jnp.float32),
        pltpu.VMEM((HIST_ROWS, D), jnp.float32),
        pltpu.VMEM((G_SEG, GRP), jnp.int32),
        pltpu.VMEM((HIST_ROWS,), jnp.int32),
    ],
)
def _sc_degree(dst_hbm, zeros_hbm, out_hbm, deg_sh, hist_v, dst_v, idx_v):
    c = lax.axis_index("c")
    s = lax.axis_index("s")
    w = s * NC + c
    zrows = pl.ds(s * 8, 8)

    @pl.when(s < HIST_ROWS // 8)
    def _():
        pltpu.sync_copy(zeros_hbm.at[zrows], deg_sh.at[zrows])

    pltpu.sync_copy(zeros_hbm.at[pl.ds(0, HIST_ROWS)], hist_v)

    @pl.loop(0, HIST_ROWS, step=16)
    def _(i):
        idx_v[pl.ds(i, 16)] = lax.iota(jnp.int32, 16) + i

    plsc.subcore_barrier()

    ones16 = jnp.ones((16,), jnp.float32)

    @pl.loop(0, N_SEG)
    def _(seg):
        pltpu.sync_copy(dst_hbm.at[pl.ds(w * G_PER_W + seg * G_SEG, G_SEG)],
                        dst_v)

        @pl.loop(0, G_SEG)
        def _(g):
            @pl.loop(0, GRP, step=16)
            def _(j):
                d16 = dst_v[g, pl.ds(j, 16)]
                plsc.addupdate_scatter(
                    hist_v,
                    [jnp.right_shift(d16, 7), jnp.bitwise_and(d16, 127)],
                    ones16)

    pltpu.sync_copy(hist_v, deg_sh.at[idx_v], add=True)
    plsc.subcore_barrier()

    @pl.when(s < HIST_ROWS // 8)
    def _():
        pltpu.sync_copy(deg_sh.at[zrows], out_hbm.at[c].at[zrows])


def _invdeg_body(deg_ref, o_ref):
    d = deg_ref[0] + deg_ref[1]
    o_ref[...] = 1.0 / jnp.maximum(d, 1.0)


def _tc_invdeg(deg):
    return pl.pallas_call(
        _invdeg_body,
        out_shape=jax.ShapeDtypeStruct((HIST_ROWS, D), jnp.float32),
    )(deg)


_BLK = 400


def _linear_relu_body(x_ref, w_ref, b_ref, o_ref):
    acc = jnp.dot(x_ref[...], w_ref[...], preferred_element_type=jnp.float32)
    o_ref[...] = jnp.maximum(acc + b_ref[...], 0.0)


def _tc_linear_relu(x, w, b):
    return pl.pallas_call(
        _linear_relu_body,
        grid=(N // _BLK,),
        in_specs=[
            pl.BlockSpec((_BLK, D), lambda i: (i, 0)),
            pl.BlockSpec((D, D), lambda i: (0, 0)),
            pl.BlockSpec((1, D), lambda i: (0, 0)),
        ],
        out_specs=pl.BlockSpec((_BLK, D), lambda i: (i, 0)),
        out_shape=jax.ShapeDtypeStruct((N, D), jnp.float32),
    )(x, w, b.reshape(1, D))


def _combine_body(p_ref, inv_ref, w_ref, b_ref, o_ref):
    p = p_ref[0] + p_ref[1]
    acc = jnp.dot(p * inv_ref[...], w_ref[...],
                  preferred_element_type=jnp.float32)
    o_ref[...] = jnp.maximum(acc + b_ref[...], 0.0)


def _tc_combine(parts, inv_deg, w, b):
    return pl.pallas_call(
        _combine_body,
        grid=(N // _BLK,),
        in_specs=[
            pl.BlockSpec((NC, _BLK, D), lambda i: (0, i, 0)),
            pl.BlockSpec((_BLK, 1), lambda i: (i, 0)),
            pl.BlockSpec((D, D), lambda i: (0, 0)),
            pl.BlockSpec((1, D), lambda i: (0, 0)),
        ],
        out_specs=pl.BlockSpec((_BLK, D), lambda i: (i, 0)),
        out_shape=jax.ShapeDtypeStruct((N, D), jnp.float32),
    )(parts, inv_deg, w, b.reshape(1, D))


def _lgamma(z):
    w = z + 7.0
    lprod = (jnp.log(z) + jnp.log(z + 1.0) + jnp.log(z + 2.0)
             + jnp.log(z + 3.0) + jnp.log(z + 4.0) + jnp.log(z + 5.0)
             + jnp.log(z + 6.0))
    wi = 1.0 / w
    wi2 = wi * wi
    stir = (w - 0.5) * jnp.log(w) - w + 0.91893853320467274178
    corr = wi * (1.0 / 12.0 - wi2 * (1.0 / 360.0 - wi2 * (1.0 / 1260.0)))
    return stir + corr - lprod


def _softplus(x):
    return jnp.maximum(x, 0.0) + jnp.log(1.0 + jnp.exp(-jnp.abs(x)))


def _heads_body(h_ref, wa1_ref, ba1_ref, wa2_ref, ba2_ref,
                wc1_ref, bc1_ref, wc2_ref, bc2_ref,
                act_ref, lp_ref, val_ref):
    h = h_ref[...]
    pooled = jnp.sum(h, axis=0, keepdims=True) * (1.0 / N)
    dev = h[0:1000]
    a1 = jnp.maximum(
        jnp.dot(dev, wa1_ref[...], preferred_element_type=jnp.float32)
        + ba1_ref[...], 0.0)
    raw = jnp.dot(a1, wa2_ref[...], preferred_element_type=jnp.float32) + ba2_ref[...]
    conc = _softplus(raw) + 1.0
    csum = jnp.sum(conc, axis=-1, keepdims=True)
    action = conc / csum
    act_ref[...] = action
    lp = (jnp.sum((conc - 1.0) * jnp.log(action))
          + jnp.sum(_lgamma(csum)) - jnp.sum(_lgamma(conc)))
    lp_ref[...] = jnp.reshape(lp, (1, 1))
    v1 = jnp.maximum(
        jnp.dot(pooled, wc1_ref[...], preferred_element_type=jnp.float32)
        + bc1_ref[...], 0.0)
    val_ref[...] = (jnp.dot(v1, wc2_ref[...], preferred_element_type=jnp.float32)
                    + bc2_ref[...])


def _tc_heads(h, wa1, ba1, wa2, ba2, wc1, bc1, wc2, bc2):
    return pl.pallas_call(
        _heads_body,
        out_shape=[
            jax.ShapeDtypeStruct((1000, AD), jnp.float32),
            jax.ShapeDtypeStruct((1, 1), jnp.float32),
            jax.ShapeDtypeStruct((1, 1), jnp.float32),
        ],
    )(h, wa1, ba1.reshape(1, D), wa2, ba2.reshape(1, AD),
      wc1, bc1.reshape(1, D), wc2, bc2.reshape(1, 1))


def kernel(x, edge_index, W_in, b_in, W_layers, b_layers,
           W_a1, b_a1, W_a2, b_a2, W_c1, b_c1, W_c2, b_c2):
    e = edge_index.shape[1]
    pad = E_PAD - e
    src_p = jnp.concatenate(
        [edge_index[0], jnp.zeros((pad,), jnp.int32)]).reshape(E_PAD // GRP, GRP)
    dst_p = jnp.concatenate(
        [edge_index[1], jnp.full((pad,), N, jnp.int32)]).reshape(E_PAD // GRP, GRP)
    zeros_d = jnp.zeros((N_PAD, D), jnp.float32)

    deg = _sc_degree(dst_p, zeros_d)
    inv_deg = _tc_invdeg(deg).reshape(N_PAD, 1)
    h = _tc_linear_relu(x, W_in, b_in)
    for l in range(3):
        parts = _sc_segment_sum(h, src_p, dst_p, zeros_d)
        h = _tc_combine(parts, inv_deg[:N], W_layers[l], b_layers[l])
    action, lp, val = _tc_heads(h, W_a1, b_a1, W_a2, b_a2,
                                W_c1, b_c1, W_c2, b_c2)
    return action, lp.reshape(()), val.reshape(())
